# baseline (speedup 1.0000x reference)
import functools

import jax
import jax.numpy as jnp
from jax import lax
from jax.experimental import pallas as pl
from jax.experimental.pallas import tpu as pltpu
from jax.experimental.pallas import tpu_sc as plsc

NUM_CLASSES = 1_000_000
HIDDEN = 64
BATCH = 16384
NUM_CORES = 2
NUM_SUBCORES = 16
NUM_WORKERS = NUM_CORES * NUM_SUBCORES
B_PER_W = BATCH // NUM_WORKERS
NUM_GROUPS = (NUM_CLASSES + 127) // 128
G_PER_W = (NUM_GROUPS + NUM_WORKERS - 1) // NUM_WORKERS
OWN_CAP = 784
DEPTH = 3
CROWS = 128

_mesh = plsc.VectorSubcoreMesh(core_axis_name="c", subcore_axis_name="s")


@functools.partial(
    pl.kernel,
    mesh=_mesh,
    out_type=jax.ShapeDtypeStruct((BATCH, 1, 128), jnp.float32),
    scratch_types=[
        pltpu.VMEM((OWN_CAP + 16,), jnp.int32),
        pltpu.VMEM((OWN_CAP + 16,), jnp.int32),
        pltpu.VMEM((OWN_CAP + 16,), jnp.int32),
        pltpu.VMEM((256,), jnp.int32),
        pltpu.VMEM((272,), jnp.int32),
        pltpu.VMEM((DEPTH, 8, 8, 128), jnp.float32),
        pltpu.VMEM((OWN_CAP, 1, 128), jnp.float32),
        pltpu.SemaphoreType.DMA,
        pltpu.SemaphoreType.DMA,
        pltpu.SemaphoreType.DMA,
        pltpu.SemaphoreType.DMA,
    ],
    compiler_params=pltpu.CompilerParams(needs_layout_passes=False),
)
def _sc_gather_rows(
    labels_hbm, table_hbm, rows_hbm,
    own_g, own_pv, mem_pv, bitmap, glist, col_v, rowst, s0, s1, s2, srow,
):
    wid = lax.axis_index("s") * NUM_CORES + lax.axis_index("c")
    g_lo = wid * G_PER_W
    g_hi = jnp.minimum(g_lo + G_PER_W, NUM_GROUPS)
    lanes = lax.iota(jnp.int32, 16)
    zeros16 = jnp.zeros((16,), jnp.int32)
    fsems = [s0, s1, s2]

    pltpu.sync_copy(labels_hbm, rowst.at[pl.ds(0, BATCH // 128)])
    for t in range(16):
        bitmap[pl.ds(t * 16, 16)] = zeros16

    def scan(i, cur):
        lab_f = rowst[i >> 3, 0, pl.ds((i & 7) * 16, 16)]
        lab = plsc.bitcast(lab_f, jnp.int32)
        g = lab >> 7
        mask = (g >= g_lo) & (g < g_hi)
        pos = i * 16 + lanes
        pv = (pos << 7) | (lab & 127)
        plsc.store_compressed(own_g.at[pl.ds(cur, 16)], g, mask=mask)
        plsc.store_compressed(own_pv.at[pl.ds(cur, 16)], pv, mask=mask)
        slot = jnp.clip(g - g_lo, 0, 255)
        plsc.store_scatter(bitmap, [slot], jnp.ones((16,), jnp.int32), mask=mask)
        return cur + plsc.all_reduce_population_count(mask)[0]

    cnt = lax.fori_loop(0, BATCH // 16, scan, jnp.int32(0))
    own_g[pl.ds(cnt, 16)] = jnp.full((16,), -1, jnp.int32)

    def compress(t, gcur):
        chunk = bitmap[pl.ds(t * 16, 16)]
        mask = chunk > 0
        plsc.store_compressed(glist.at[pl.ds(gcur, 16)], g_lo + t * 16 + lanes, mask=mask)
        return gcur + plsc.all_reduce_population_count(mask)[0]

    gcnt = lax.fori_loop(0, 16, compress, jnp.int32(0))
    gmax = jnp.maximum(gcnt - 1, 0)

    def fire(idx, slot):
        gi = plsc.load_gather(
            glist, [jnp.broadcast_to(jnp.minimum(idx, gmax), (16,)).astype(jnp.int32)]
        )
        gc = jnp.clip(gi[0], 0, NUM_GROUPS - 1)
        pltpu.async_copy(
            table_hbm.at[:, :, pl.ds(pl.multiple_of(gc * 128, 128), 128)],
            col_v.at[slot], fsems[slot],
        )

    for k in range(DEPTH):
        fire(jnp.int32(k), k)

    kchunks = (cnt + 15) >> 4

    def process(idx, slot, rowidx):
        pltpu.make_async_copy(
            table_hbm.at[:, :, pl.ds(0, 128)], col_v.at[slot], fsems[slot]
        ).wait()
        gi_sp = plsc.load_gather(
            glist, [jnp.broadcast_to(jnp.minimum(idx, gmax), (16,)).astype(jnp.int32)]
        )
        slot_sp = jnp.full((16,), slot, jnp.int32)

        def mscan(k, mcur):
            chunk = own_g[pl.ds(k * 16, 16)]
            mask = chunk == gi_sp
            pvc = own_pv[pl.ds(k * 16, 16)]
            plsc.store_compressed(mem_pv.at[pl.ds(mcur, 16)], pvc, mask=mask)
            return mcur + plsc.all_reduce_population_count(mask)[0]

        mcnt = lax.fori_loop(0, kchunks, mscan, jnp.int32(0))

        def member(m, ridx):
            pv_sp = plsc.load_gather(mem_pv, [jnp.broadcast_to(m, (16,)).astype(jnp.int32)])
            m_sp = pv_sp & 127
            p = jnp.clip(pv_sp[0] >> 7, 0, BATCH - 1)
            for c in range(4):
                r_ids = (c * 16 + lanes) >> 3
                h8_ids = (c * 16 + lanes) & 7
                vals = plsc.load_gather(col_v, [slot_sp, r_ids, h8_ids, m_sp])
                rowst[ridx, 0, pl.ds(c * 16, 16)] = vals
            pltpu.async_copy(rowst.at[pl.ds(ridx, 1)], rows_hbm.at[pl.ds(p, 1)], srow)
            return ridx + 1

        rowidx = lax.fori_loop(0, mcnt, member, rowidx)
        fire(idx + DEPTH, slot)
        return rowidx

    def per_round(it, carry):
        rowidx = carry
        for k in range(DEPTH):
            rowidx = process(it * DEPTH + k, k, rowidx)
        return rowidx

    nrounds = (gcnt + DEPTH - 1) // DEPTH
    total_rows = lax.fori_loop(0, nrounds, per_round, jnp.int32(0))

    for k in range(DEPTH):
        pltpu.make_async_copy(
            table_hbm.at[:, :, pl.ds(0, 128)], col_v.at[k], fsems[k]
        ).wait()

    def drain(m, carry):
        pltpu.make_async_copy(
            rows_hbm.at[pl.ds(0, 1)], rowst.at[pl.ds(0, 1)], srow
        ).wait()
        return carry

    lax.fori_loop(0, total_rows, drain, jnp.int32(0))


@functools.partial(
    pl.kernel,
    mesh=_mesh,
    out_type=jax.ShapeDtypeStruct((8, 8, BATCH), jnp.float32),
    scratch_types=[
        pltpu.VMEM((2, CROWS, 1, 128), jnp.float32),
        pltpu.VMEM((8, 8, B_PER_W), jnp.float32),
        pltpu.SemaphoreType.DMA,
        pltpu.SemaphoreType.DMA,
    ],
    compiler_params=pltpu.CompilerParams(needs_layout_passes=False),
)
def _sc_transpose(rows_hbm, outt_hbm, loc_v, stage_v, sem_a, sem_b):
    wid = lax.axis_index("s") * NUM_CORES + lax.axis_index("c")
    base = pl.multiple_of(wid * B_PER_W, B_PER_W)
    lanes = lax.iota(jnp.int32, 16)
    z_sp = jnp.zeros((16,), jnp.int32)
    csems = [sem_a, sem_b]
    NCHK = B_PER_W // CROWS

    def cfire(c, slot):
        cc = jnp.minimum(c, NCHK - 1)
        pltpu.async_copy(
            rows_hbm.at[pl.ds(base + cc * CROWS, CROWS)], loc_v.at[slot], csems[slot]
        )

    for k in range(2):
        cfire(jnp.int32(k), k)

    def chunk(c, slot):
        pltpu.make_async_copy(
            rows_hbm.at[pl.ds(0, CROWS)], loc_v.at[slot], csems[slot]
        ).wait()
        slot_sp = jnp.full((16,), slot, jnp.int32)
        cbase = jnp.minimum(c, NCHK - 1) * CROWS

        def per_j(j, carry):
            pos_ids = j * 16 + lanes
            for h0 in range(0, HIDDEN, 16):
                for d in range(16):
                    h_ids = h0 + ((lanes + d) & 15)
                    vals = plsc.load_gather(loc_v, [slot_sp, pos_ids, z_sp, h_ids])
                    plsc.store_scatter(
                        stage_v, [h_ids >> 3, h_ids & 7, cbase + pos_ids], vals
                    )
            return carry

        lax.fori_loop(0, CROWS // 16, per_j, 0)
        cfire(c + 2, slot)

    def round_(it, carry):
        for k in range(2):
            chunk(it * 2 + k, k)
        return carry

    lax.fori_loop(0, NCHK // 2, round_, 0)
    for k in range(2):
        pltpu.make_async_copy(
            rows_hbm.at[pl.ds(0, CROWS)], loc_v.at[k], csems[k]
        ).wait()
    pltpu.sync_copy(stage_v, outt_hbm.at[:, :, pl.ds(base, B_PER_W)])


def kernel(labels, embedding_table):
    table3 = embedding_table.T.reshape(8, 8, NUM_CLASSES)
    labels_f = lax.bitcast_convert_type(labels.astype(jnp.int32), jnp.float32)
    labels3 = labels_f.reshape(BATCH // 128, 1, 128)
    rows = _sc_gather_rows(labels3, table3)
    outt = _sc_transpose(rows)
    return outt.reshape(HIDDEN, BATCH).T


# speedup vs baseline: 1.0925x; 1.0925x over previous
import functools

import jax
import jax.numpy as jnp
from jax import lax
from jax.experimental import pallas as pl
from jax.experimental.pallas import tpu as pltpu
from jax.experimental.pallas import tpu_sc as plsc

NUM_CLASSES = 1_000_000
HIDDEN = 64
BATCH = 16384
NUM_CORES = 2
NUM_SUBCORES = 16
NUM_WORKERS = NUM_CORES * NUM_SUBCORES
B_PER_W = BATCH // NUM_WORKERS
NUM_GROUPS = (NUM_CLASSES + 127) // 128
G_PER_W = (NUM_GROUPS + NUM_WORKERS - 1) // NUM_WORKERS
OWN_CAP = 784
DEPTH = 3
CROWS = 128

_mesh = plsc.VectorSubcoreMesh(core_axis_name="c", subcore_axis_name="s")


@functools.partial(
    pl.kernel,
    mesh=_mesh,
    out_type=jax.ShapeDtypeStruct((BATCH, 1, 128), jnp.float32),
    scratch_types=[
        pltpu.VMEM((OWN_CAP + 16,), jnp.int32),
        pltpu.VMEM((OWN_CAP + 16,), jnp.int32),
        pltpu.VMEM((OWN_CAP + 16,), jnp.int32),
        pltpu.VMEM((OWN_CAP + 16,), jnp.int32),
        pltpu.VMEM((128,), jnp.int32),
        pltpu.VMEM((32,), jnp.int32),
        pltpu.VMEM((256,), jnp.int32),
        pltpu.VMEM((272,), jnp.int32),
        pltpu.VMEM((DEPTH, 8, 8, 128), jnp.float32),
        pltpu.VMEM((OWN_CAP, 1, 128), jnp.float32),
        pltpu.SemaphoreType.DMA,
        pltpu.SemaphoreType.DMA,
        pltpu.SemaphoreType.DMA,
        pltpu.SemaphoreType.DMA,
    ],
    compiler_params=pltpu.CompilerParams(needs_layout_passes=False),
)
def _sc_gather_rows(
    labels_hbm, table_hbm, rows_hbm,
    own_g, own_pv, sort_g, sort_pv, mem_pv, bstart, bitmap, glist, col_v, rowst, s0, s1, s2, srow,
):
    wid = lax.axis_index("s") * NUM_CORES + lax.axis_index("c")
    g_lo = wid * G_PER_W
    g_hi = jnp.minimum(g_lo + G_PER_W, NUM_GROUPS)
    lanes = lax.iota(jnp.int32, 16)
    zeros16 = jnp.zeros((16,), jnp.int32)
    fsems = [s0, s1, s2]

    pltpu.sync_copy(labels_hbm, rowst.at[pl.ds(0, BATCH // 128)])
    for t in range(16):
        bitmap[pl.ds(t * 16, 16)] = zeros16

    def scan(i, cur):
        lab_f = rowst[i >> 3, 0, pl.ds((i & 7) * 16, 16)]
        lab = plsc.bitcast(lab_f, jnp.int32)
        g = lab >> 7
        mask = (g >= g_lo) & (g < g_hi)
        pos = i * 16 + lanes
        pv = (pos << 7) | (lab & 127)
        plsc.store_compressed(own_g.at[pl.ds(cur, 16)], g, mask=mask)
        plsc.store_compressed(own_pv.at[pl.ds(cur, 16)], pv, mask=mask)
        slot = jnp.clip(g - g_lo, 0, 255)
        plsc.store_scatter(bitmap, [slot], jnp.ones((16,), jnp.int32), mask=mask)
        return cur + plsc.all_reduce_population_count(mask)[0]

    cnt = lax.fori_loop(0, BATCH // 16, scan, jnp.int32(0))
    own_g[pl.ds(cnt, 16)] = jnp.full((16,), -1, jnp.int32)
    kchunks0 = (cnt + 15) >> 4

    bcur = jnp.int32(0)
    bst_parts = []
    for b in range(16):
        b_sp = jnp.full((16,), b, jnp.int32)
        bst_parts.append(jnp.where(lanes == b, jnp.broadcast_to(bcur, (16,)), 0))

        def bscan(k, cur2, b_sp=b_sp):
            chunk = own_g[pl.ds(k * 16, 16)]
            mask = ((chunk - g_lo) >> 4) == b_sp
            mask = mask & (chunk >= 0)
            pvc = own_pv[pl.ds(k * 16, 16)]
            plsc.store_compressed(sort_g.at[pl.ds(cur2, 16)], chunk, mask=mask)
            plsc.store_compressed(sort_pv.at[pl.ds(cur2, 16)], pvc, mask=mask)
            return cur2 + plsc.all_reduce_population_count(mask)[0]

        bcur = lax.fori_loop(0, kchunks0, bscan, bcur)
    bst_vec = bst_parts[0]
    for part in bst_parts[1:]:
        bst_vec = bst_vec | part
    bstart[pl.ds(0, 16)] = bst_vec
    bstart[pl.ds(16, 16)] = jnp.broadcast_to(bcur, (16,))
    sort_g[pl.ds(bcur, 16)] = jnp.full((16,), -1, jnp.int32)

    def compress(t, gcur):
        chunk = bitmap[pl.ds(t * 16, 16)]
        mask = chunk > 0
        plsc.store_compressed(glist.at[pl.ds(gcur, 16)], g_lo + t * 16 + lanes, mask=mask)
        return gcur + plsc.all_reduce_population_count(mask)[0]

    gcnt = lax.fori_loop(0, 16, compress, jnp.int32(0))
    gmax = jnp.maximum(gcnt - 1, 0)

    def fire(idx, slot):
        gi = plsc.load_gather(
            glist, [jnp.broadcast_to(jnp.minimum(idx, gmax), (16,)).astype(jnp.int32)]
        )
        gc = jnp.clip(gi[0], 0, NUM_GROUPS - 1)
        pltpu.async_copy(
            table_hbm.at[:, :, pl.ds(pl.multiple_of(gc * 128, 128), 128)],
            col_v.at[slot], fsems[slot],
        )

    for k in range(DEPTH):
        fire(jnp.int32(k), k)

    kchunks = (cnt + 15) >> 4

    def process(idx, slot, rowidx):
        pltpu.make_async_copy(
            table_hbm.at[:, :, pl.ds(0, 128)], col_v.at[slot], fsems[slot]
        ).wait()
        gi_sp = plsc.load_gather(
            glist, [jnp.broadcast_to(jnp.minimum(idx, gmax), (16,)).astype(jnp.int32)]
        )
        slot_sp = jnp.full((16,), slot, jnp.int32)
        bidx = (jnp.clip(gi_sp[0], g_lo, g_hi - 1) - g_lo) >> 4
        bs = plsc.load_gather(bstart, [jnp.broadcast_to(bidx, (16,)).astype(jnp.int32)])[0]
        be = plsc.load_gather(
            bstart, [jnp.broadcast_to(bidx + 1, (16,)).astype(jnp.int32)]
        )[0]

        def mscan(k, mcur):
            chunk = sort_g[pl.ds(k * 16, 16)]
            mask = chunk == gi_sp
            pvc = sort_pv[pl.ds(k * 16, 16)]
            plsc.store_compressed(mem_pv.at[pl.ds(mcur, 16)], pvc, mask=mask)
            return mcur + plsc.all_reduce_population_count(mask)[0]

        mcnt = lax.fori_loop(bs >> 4, (be + 15) >> 4, mscan, jnp.int32(0))

        def member(m, ridx):
            pv_sp = plsc.load_gather(mem_pv, [jnp.broadcast_to(m, (16,)).astype(jnp.int32)])
            m_sp = pv_sp & 127
            p = jnp.clip(pv_sp[0] >> 7, 0, BATCH - 1)
            for c in range(4):
                r_ids = (c * 16 + lanes) >> 3
                h8_ids = (c * 16 + lanes) & 7
                vals = plsc.load_gather(col_v, [slot_sp, r_ids, h8_ids, m_sp])
                rowst[ridx, 0, pl.ds(c * 16, 16)] = vals
            pltpu.async_copy(rowst.at[pl.ds(ridx, 1)], rows_hbm.at[pl.ds(p, 1)], srow)
            return ridx + 1

        rowidx = lax.fori_loop(0, mcnt, member, rowidx)
        fire(idx + DEPTH, slot)
        return rowidx

    def per_round(it, carry):
        rowidx = carry
        for k in range(DEPTH):
            rowidx = process(it * DEPTH + k, k, rowidx)
        return rowidx

    nrounds = (gcnt + DEPTH - 1) // DEPTH
    total_rows = lax.fori_loop(0, nrounds, per_round, jnp.int32(0))

    for k in range(DEPTH):
        pltpu.make_async_copy(
            table_hbm.at[:, :, pl.ds(0, 128)], col_v.at[k], fsems[k]
        ).wait()

    def drain(m, carry):
        pltpu.make_async_copy(
            rows_hbm.at[pl.ds(0, 1)], rowst.at[pl.ds(0, 1)], srow
        ).wait()
        return carry

    lax.fori_loop(0, total_rows, drain, jnp.int32(0))


@functools.partial(
    pl.kernel,
    mesh=_mesh,
    out_type=jax.ShapeDtypeStruct((8, 8, BATCH), jnp.float32),
    scratch_types=[
        pltpu.VMEM((B_PER_W, 1, 128), jnp.float32),
        pltpu.VMEM((8, 8, B_PER_W), jnp.float32),
        pltpu.SemaphoreType.DMA,
    ],
    compiler_params=pltpu.CompilerParams(needs_layout_passes=False),
)
def _sc_transpose(rows_hbm, outt_hbm, loc_v, stage_v, sem):
    wid = lax.axis_index("s") * NUM_CORES + lax.axis_index("c")
    base = pl.multiple_of(wid * B_PER_W, B_PER_W)
    pltpu.sync_copy(rows_hbm.at[pl.ds(base, B_PER_W)], loc_v)
    lanes = lax.iota(jnp.int32, 16)
    z_sp = jnp.zeros((16,), jnp.int32)

    def per_j(j, carry):
        pos_ids = j * 16 + lanes
        for h0 in range(0, HIDDEN, 16):
            for d in range(16):
                h_ids = h0 + ((lanes + d) & 15)
                vals = plsc.load_gather(loc_v, [pos_ids, z_sp, h_ids])
                plsc.store_scatter(stage_v, [h_ids >> 3, h_ids & 7, pos_ids], vals)
        return carry

    lax.fori_loop(0, B_PER_W // 16, per_j, 0)
    pltpu.sync_copy(stage_v, outt_hbm.at[:, :, pl.ds(base, B_PER_W)])


def kernel(labels, embedding_table):
    table3 = embedding_table.T.reshape(8, 8, NUM_CLASSES)
    labels_f = lax.bitcast_convert_type(labels.astype(jnp.int32), jnp.float32)
    labels3 = labels_f.reshape(BATCH // 128, 1, 128)
    rows = _sc_gather_rows(labels3, table3)
    outt = _sc_transpose(rows)
    return outt.reshape(HIDDEN, BATCH).T


# speedup vs baseline: 1.1975x; 1.0962x over previous
import functools

import jax
import jax.numpy as jnp
from jax import lax
from jax.experimental import pallas as pl
from jax.experimental.pallas import tpu as pltpu
from jax.experimental.pallas import tpu_sc as plsc

NUM_CLASSES = 1_000_000
HIDDEN = 64
BATCH = 16384
NUM_CORES = 2
NUM_SUBCORES = 16
NUM_WORKERS = NUM_CORES * NUM_SUBCORES
B_PER_W = BATCH // NUM_WORKERS
NUM_GROUPS = (NUM_CLASSES + 127) // 128
G_PER_W = (NUM_GROUPS + NUM_WORKERS - 1) // NUM_WORKERS
OWN_CAP = 784
DEPTH = 3
CROWS = 128

_mesh = plsc.VectorSubcoreMesh(core_axis_name="c", subcore_axis_name="s")


@functools.partial(
    pl.kernel,
    mesh=_mesh,
    out_type=jax.ShapeDtypeStruct((BATCH, 1, 128), jnp.float32),
    scratch_types=[
        pltpu.VMEM((OWN_CAP + 16,), jnp.int32),
        pltpu.VMEM((OWN_CAP + 16,), jnp.int32),
        pltpu.VMEM((OWN_CAP + 16,), jnp.int32),
        pltpu.VMEM((OWN_CAP + 16,), jnp.int32),
        pltpu.VMEM((128,), jnp.int32),
        pltpu.VMEM((32,), jnp.int32),
        pltpu.VMEM((256,), jnp.int32),
        pltpu.VMEM((272,), jnp.int32),
        pltpu.VMEM((DEPTH, 8, 8, 128), jnp.float32),
        pltpu.VMEM((OWN_CAP, 1, 128), jnp.float32),
        pltpu.SemaphoreType.DMA,
        pltpu.SemaphoreType.DMA,
        pltpu.SemaphoreType.DMA,
        pltpu.SemaphoreType.DMA,
    ],
    compiler_params=pltpu.CompilerParams(needs_layout_passes=False),
)
def _sc_gather_rows(
    labels_hbm, table_hbm, rows_hbm,
    own_g, own_pv, sort_g, sort_pv, mem_pv, bstart, bitmap, glist, col_v, rowst, s0, s1, s2, srow,
):
    wid = lax.axis_index("s") * NUM_CORES + lax.axis_index("c")
    g_lo = wid * G_PER_W
    g_hi = jnp.minimum(g_lo + G_PER_W, NUM_GROUPS)
    lanes = lax.iota(jnp.int32, 16)
    zeros16 = jnp.zeros((16,), jnp.int32)
    fsems = [s0, s1, s2]

    pltpu.sync_copy(labels_hbm, rowst.at[pl.ds(0, BATCH // 128)])
    for t in range(16):
        bitmap[pl.ds(t * 16, 16)] = zeros16

    def scan(i, cur):
        lab_f = rowst[i >> 3, 0, pl.ds((i & 7) * 16, 16)]
        lab = plsc.bitcast(lab_f, jnp.int32)
        g = lab >> 7
        mask = (g >= g_lo) & (g < g_hi)
        pos = i * 16 + lanes
        pv = (pos << 7) | (lab & 127)
        plsc.store_compressed(own_g.at[pl.ds(cur, 16)], g, mask=mask)
        plsc.store_compressed(own_pv.at[pl.ds(cur, 16)], pv, mask=mask)
        slot = jnp.clip(g - g_lo, 0, 255)
        plsc.store_scatter(bitmap, [slot], jnp.ones((16,), jnp.int32), mask=mask)
        return cur + plsc.all_reduce_population_count(mask)[0]

    cnt = lax.fori_loop(0, BATCH // 16, scan, jnp.int32(0))
    own_g[pl.ds(cnt, 16)] = jnp.full((16,), -1, jnp.int32)
    kchunks0 = (cnt + 15) >> 4

    bcur = jnp.int32(0)
    bst_parts = []
    for b in range(16):
        b_sp = jnp.full((16,), b, jnp.int32)
        bst_parts.append(jnp.where(lanes == b, jnp.broadcast_to(bcur, (16,)), 0))

        def bscan(k, cur2, b_sp=b_sp):
            chunk = own_g[pl.ds(k * 16, 16)]
            mask = ((chunk - g_lo) >> 4) == b_sp
            mask = mask & (chunk >= 0)
            pvc = own_pv[pl.ds(k * 16, 16)]
            plsc.store_compressed(sort_g.at[pl.ds(cur2, 16)], chunk, mask=mask)
            plsc.store_compressed(sort_pv.at[pl.ds(cur2, 16)], pvc, mask=mask)
            return cur2 + plsc.all_reduce_population_count(mask)[0]

        bcur = lax.fori_loop(0, kchunks0, bscan, bcur)
    bst_vec = bst_parts[0]
    for part in bst_parts[1:]:
        bst_vec = bst_vec | part
    bstart[pl.ds(0, 16)] = bst_vec
    bstart[pl.ds(16, 16)] = jnp.broadcast_to(bcur, (16,))
    sort_g[pl.ds(bcur, 16)] = jnp.full((16,), -1, jnp.int32)

    def compress(t, gcur):
        chunk = bitmap[pl.ds(t * 16, 16)]
        mask = chunk > 0
        plsc.store_compressed(glist.at[pl.ds(gcur, 16)], g_lo + t * 16 + lanes, mask=mask)
        return gcur + plsc.all_reduce_population_count(mask)[0]

    gcnt = lax.fori_loop(0, 16, compress, jnp.int32(0))
    gmax = jnp.maximum(gcnt - 1, 0)

    def fire(idx, slot):
        gi = plsc.load_gather(
            glist, [jnp.broadcast_to(jnp.minimum(idx, gmax), (16,)).astype(jnp.int32)]
        )
        gc = jnp.clip(gi[0], 0, NUM_GROUPS - 1)
        pltpu.async_copy(
            table_hbm.at[:, :, pl.ds(pl.multiple_of(gc * 128, 128), 128)],
            col_v.at[slot], fsems[slot],
        )

    for k in range(DEPTH):
        fire(jnp.int32(k), k)

    kchunks = (cnt + 15) >> 4

    def process(idx, slot, rowidx):
        pltpu.make_async_copy(
            table_hbm.at[:, :, pl.ds(0, 128)], col_v.at[slot], fsems[slot]
        ).wait()
        gi_sp = plsc.load_gather(
            glist, [jnp.broadcast_to(jnp.minimum(idx, gmax), (16,)).astype(jnp.int32)]
        )
        slot_sp = jnp.full((16,), slot, jnp.int32)
        bidx = (jnp.clip(gi_sp[0], g_lo, g_hi - 1) - g_lo) >> 4
        bs = plsc.load_gather(bstart, [jnp.broadcast_to(bidx, (16,)).astype(jnp.int32)])[0]
        be = plsc.load_gather(
            bstart, [jnp.broadcast_to(bidx + 1, (16,)).astype(jnp.int32)]
        )[0]

        def mscan(k, mcur):
            chunk = sort_g[pl.ds(k * 16, 16)]
            mask = chunk == gi_sp
            pvc = sort_pv[pl.ds(k * 16, 16)]
            plsc.store_compressed(mem_pv.at[pl.ds(mcur, 16)], pvc, mask=mask)
            return mcur + plsc.all_reduce_population_count(mask)[0]

        mcnt = lax.fori_loop(bs >> 4, (be + 15) >> 4, mscan, jnp.int32(0))

        def member(m, ridx):
            pv_sp = plsc.load_gather(mem_pv, [jnp.broadcast_to(m, (16,)).astype(jnp.int32)])
            m_sp = pv_sp & 127
            p = jnp.clip(pv_sp[0] >> 7, 0, BATCH - 1)
            for c in range(4):
                r_ids = (c * 16 + lanes) >> 3
                h8_ids = (c * 16 + lanes) & 7
                vals = plsc.load_gather(col_v, [slot_sp, r_ids, h8_ids, m_sp])
                rowst[ridx, 0, pl.ds(c * 16, 16)] = vals
            pltpu.async_copy(rowst.at[pl.ds(ridx, 1)], rows_hbm.at[pl.ds(p, 1)], srow)
            return ridx + 1

        rowidx = lax.fori_loop(0, mcnt, member, rowidx)
        fire(idx + DEPTH, slot)
        return rowidx

    def per_round(it, carry):
        rowidx = carry
        for k in range(DEPTH):
            rowidx = process(it * DEPTH + k, k, rowidx)
        return rowidx

    nrounds = (gcnt + DEPTH - 1) // DEPTH
    total_rows = lax.fori_loop(0, nrounds, per_round, jnp.int32(0))

    for k in range(DEPTH):
        pltpu.make_async_copy(
            table_hbm.at[:, :, pl.ds(0, 128)], col_v.at[k], fsems[k]
        ).wait()

    def drain(m, carry):
        pltpu.make_async_copy(
            rows_hbm.at[pl.ds(0, 1)], rowst.at[pl.ds(0, 1)], srow
        ).wait()
        return carry

    lax.fori_loop(0, total_rows, drain, jnp.int32(0))


@functools.partial(
    pl.kernel,
    mesh=_mesh,
    out_type=jax.ShapeDtypeStruct((8, 8, BATCH), jnp.float32),
    scratch_types=[
        pltpu.VMEM((B_PER_W, 1, 128), jnp.float32),
        pltpu.VMEM((8, 8, B_PER_W), jnp.float32),
        pltpu.SemaphoreType.DMA,
    ],
    compiler_params=pltpu.CompilerParams(needs_layout_passes=False),
)
def _sc_transpose(rows_hbm, outt_hbm, loc_v, stage_v, sem):
    wid = lax.axis_index("s") * NUM_CORES + lax.axis_index("c")
    base = pl.multiple_of(wid * B_PER_W, B_PER_W)
    pltpu.sync_copy(rows_hbm.at[pl.ds(base, B_PER_W)], loc_v)
    lanes = lax.iota(jnp.int32, 16)
    z_sp = jnp.zeros((16,), jnp.int32)

    def per_j(j, carry):
        pos_ids = j * 16 + lanes
        for h0 in range(0, HIDDEN, 16):
            for d in range(16):
                h_ids = h0 + ((lanes + d) & 15)
                vals = plsc.load_gather(loc_v, [pos_ids, z_sp, h_ids])
                plsc.store_scatter(stage_v, [h_ids >> 3, h_ids & 7, pos_ids], vals)
        return carry

    lax.fori_loop(0, B_PER_W // 16, per_j, 0)
    pltpu.sync_copy(stage_v, outt_hbm.at[:, :, pl.ds(base, B_PER_W)])


def kernel(labels, embedding_table):
    table3 = embedding_table.T.reshape(8, 8, NUM_CLASSES)
    labels_f = lax.bitcast_convert_type(labels.astype(jnp.int32), jnp.float32)
    labels3 = labels_f.reshape(BATCH // 128, 1, 128)
    rows = _sc_gather_rows(labels3, table3)
    return rows.reshape(BATCH, 128)[:, :HIDDEN]
